# Initial kernel scaffold; baseline (speedup 1.0000x reference)
import jax
import jax.numpy as jnp
from jax.experimental import pallas as pl


def kernel(g, features, weight, edge_weight, W1, b1, lin1_W, lin1_b, lin2_W, lin2_b, lin3_W, lin3_b):
    raise NotImplementedError("write your pallas kernel here")


# speedup vs baseline: 18.2256x; 18.2256x over previous
import functools

import jax
import jax.numpy as jnp
from jax import lax
from jax.experimental import pallas as pl
from jax.experimental.pallas import tpu as pltpu
from jax.experimental.pallas import tpu_sc as plsc

N = 1024
E = 32768
NC = 2
NS = 16
NW = NC * NS
EPW = E // NW
ROWS = EPW // 128
APW = (N * N) // NS


def _sc_body(src_hbm, dst_hbm, ew_hbm, a_out, deg_out,
             a_sp, din_sp, dout_sp, sv, dv, wv, xv, ov, zb, sem):
    c = lax.axis_index("c")
    s = lax.axis_index("s")
    wid = s * NC + c

    z16 = jnp.zeros((16,), jnp.float32)
    o16 = jnp.ones((16,), jnp.float32)

    def zb_body(i, _):
        zb[pl.ds(i * 16, 16)] = z16
        return 0

    lax.fori_loop(0, 8192 // 16, zb_body, 0)

    def ov_body(i, _):
        ov[pl.ds(i * 16, 16)] = o16
        return 0

    lax.fori_loop(0, 128 // 16, ov_body, 0)

    zcps = [
        pltpu.async_copy(zb, a_sp.at[pl.ds(s * APW + q * 8192, 8192)], sem)
        for q in range(APW // 8192)
    ]

    @pl.when(s == 0)
    def _():
        pltpu.sync_copy(zb.at[pl.ds(0, N)], din_sp)
        pltpu.sync_copy(zb.at[pl.ds(0, N)], dout_sp)

    pltpu.sync_copy(src_hbm.at[wid], sv)
    pltpu.sync_copy(dst_hbm.at[wid], dv)
    pltpu.sync_copy(ew_hbm.at[wid], wv)

    for j in range(ROWS):
        def x_body(k, _):
            sl = pl.ds(k * 16, 16)
            xv[j, sl] = dv[j, sl] * N + sv[j, sl]
            return 0
        lax.fori_loop(0, 128 // 16, x_body, 0)

    for cp in zcps:
        cp.wait()
    plsc.subcore_barrier()

    cps = []
    for j in range(ROWS):
        cps.append(pltpu.async_copy(wv.at[j], a_sp.at[xv.at[j]], sem, add=True))
        cps.append(pltpu.async_copy(ov, din_sp.at[dv.at[j]], sem, add=True))
        cps.append(pltpu.async_copy(ov, dout_sp.at[sv.at[j]], sem, add=True))
    for cp in cps:
        cp.wait()
    plsc.subcore_barrier()

    pltpu.sync_copy(a_sp.at[pl.ds(s * APW, APW)],
                    a_out.at[pl.ds(c * (N * N) + s * APW, APW)])

    @pl.when(s == 0)
    def _():
        pltpu.sync_copy(din_sp, deg_out.at[pl.ds(c * 2 * N, N)])
        pltpu.sync_copy(dout_sp, deg_out.at[pl.ds(c * 2 * N + N, N)])


@functools.partial(jax.jit, static_argnums=())
def _sc_build(src, dst, ew):
    mesh = plsc.VectorSubcoreMesh(core_axis_name="c", subcore_axis_name="s")
    f = pl.kernel(
        _sc_body,
        out_type=(
            jax.ShapeDtypeStruct((NC * N * N,), jnp.float32),
            jax.ShapeDtypeStruct((NC * 2 * N,), jnp.float32),
        ),
        mesh=mesh,
        scratch_types=(
            pltpu.VMEM_SHARED((N * N,), jnp.float32),
            pltpu.VMEM_SHARED((N,), jnp.float32),
            pltpu.VMEM_SHARED((N,), jnp.float32),
            pltpu.VMEM((ROWS, 128), jnp.int32),
            pltpu.VMEM((ROWS, 128), jnp.int32),
            pltpu.VMEM((ROWS, 128), jnp.float32),
            pltpu.VMEM((ROWS, 128), jnp.int32),
            pltpu.VMEM((128,), jnp.float32),
            pltpu.VMEM((8192,), jnp.float32),
            pltpu.SemaphoreType.DMA,
        ),
    )
    return f(src, dst, ew)


def _tc_body(ap_ref, din_ref, dout_ref, feat_ref, w1_ref, b1_ref, wgt_ref,
             l1w_ref, l1b_ref, l2w_ref, l2b_ref, l3w_ref, l3b_ref, out_ref):
    a = ap_ref[0] + ap_ref[1]
    ri = lax.rsqrt(jnp.maximum(din_ref[0] + din_ref[1], 1.0))
    ro = lax.rsqrt(jnp.maximum(dout_ref[0] + dout_ref[1], 1.0))
    m = a * ri * ro

    dot = functools.partial(jnp.dot, preferred_element_type=jnp.float32)
    t0 = dot(m, feat_ref[...])
    x1 = jnp.maximum(dot(t0, w1_ref[...]) + b1_ref[...], 0.0)
    t1 = dot(x1, wgt_ref[...])
    x2 = jnp.maximum(dot(m, t1), 0.0)
    x3 = jnp.maximum(dot(x2, l1w_ref[...]) + l1b_ref[...], 0.0)
    x4 = jnp.maximum(dot(x3, l2w_ref[...]) + l2b_ref[...], 0.0)
    out_ref[...] = dot(x4, l3w_ref[...]) + l3b_ref[...]


def kernel(g, features, weight, edge_weight, W1, b1, lin1_W, lin1_b,
           lin2_W, lin2_b, lin3_W, lin3_b):
    src = g[0].reshape(NW, ROWS, 128)
    dst = g[1].reshape(NW, ROWS, 128)
    ew = edge_weight.reshape(NW, ROWS, 128)

    a_flat, deg_flat = _sc_build(src, dst, ew)
    ap = a_flat.reshape(NC, N, N)
    degs = deg_flat.reshape(NC, 2, N)
    din = degs[:, 0, :].reshape(NC, N, 1)
    dout = degs[:, 1, :].reshape(NC, 1, N)

    out = pl.pallas_call(
        _tc_body,
        out_shape=jax.ShapeDtypeStruct((N, 16), jnp.float32),
    )(ap, din, dout, features, W1, b1.reshape(1, -1), weight,
      lin1_W, lin1_b.reshape(1, -1), lin2_W, lin2_b.reshape(1, -1),
      lin3_W, lin3_b.reshape(1, -1))
    return out


# speedup vs baseline: 19.7393x; 1.0831x over previous
import functools

import jax
import jax.numpy as jnp
from jax import lax
from jax.experimental import pallas as pl
from jax.experimental.pallas import tpu as pltpu
from jax.experimental.pallas import tpu_sc as plsc

N = 1024
E = 32768
NC = 2
NS = 16
NW = NC * NS
EPW = E // NW
ROWS = EPW // 128
APW = (N * N) // NS


def _sc_body(src_hbm, dst_hbm, ew_hbm, a_out, deg_out,
             a_sp, din_sp, dout_sp, sf, df, wf, sv2, dv2, xv, ov, zb, sem):
    c = lax.axis_index("c")
    s = lax.axis_index("s")
    wid = s * NC + c
    base = wid * EPW

    z16 = jnp.zeros((16,), jnp.float32)
    o16 = jnp.ones((16,), jnp.float32)

    def zb_body(i, _):
        zb[pl.ds(i * 16, 16)] = z16
        return 0

    lax.fori_loop(0, 8192 // 16, zb_body, 0)

    def ov_body(i, _):
        ov[pl.ds(i * 16, 16)] = o16
        return 0

    lax.fori_loop(0, 128 // 16, ov_body, 0)

    zcps = [
        pltpu.async_copy(zb, a_sp.at[pl.ds(s * APW + q * 8192, 8192)], sem)
        for q in range(APW // 8192)
    ]

    @pl.when(s == 0)
    def _():
        pltpu.sync_copy(zb.at[pl.ds(0, N)], din_sp)
        pltpu.sync_copy(zb.at[pl.ds(0, N)], dout_sp)

    pltpu.sync_copy(src_hbm.at[pl.ds(base, EPW)], sf)
    pltpu.sync_copy(dst_hbm.at[pl.ds(base, EPW)], df)
    pltpu.sync_copy(ew_hbm.at[pl.ds(base, EPW)], wf)

    for j in range(ROWS):
        def x_body(k, _):
            sl = pl.ds(j * 128 + k * 16, 16)
            sl2 = pl.ds(k * 16, 16)
            r = df[sl]
            col = sf[sl]
            addr = ((r >> 3) << 13) + ((col >> 7) << 10) + ((r & 7) << 7) \
                + (col & 127)
            xv[j, sl2] = addr
            dv2[j, sl2] = r
            sv2[j, sl2] = col
            return 0
        lax.fori_loop(0, 128 // 16, x_body, 0)

    for cp in zcps:
        cp.wait()
    plsc.subcore_barrier()

    cps = []
    for j in range(ROWS):
        cps.append(pltpu.async_copy(
            wf.at[pl.ds(j * 128, 128)], a_sp.at[xv.at[j]], sem, add=True))
        cps.append(pltpu.async_copy(ov, din_sp.at[dv2.at[j]], sem, add=True))
        cps.append(pltpu.async_copy(ov, dout_sp.at[sv2.at[j]], sem, add=True))
    for cp in cps:
        cp.wait()
    plsc.subcore_barrier()

    pltpu.sync_copy(a_sp.at[pl.ds(s * APW, APW)],
                    a_out.at[pl.ds(c * (N * N) + s * APW, APW)])

    @pl.when(s == 0)
    def _():
        pltpu.sync_copy(din_sp, deg_out.at[pl.ds(c * 2 * N, N)])
        pltpu.sync_copy(dout_sp, deg_out.at[pl.ds(c * 2 * N + N, N)])


def _sc_build(src, dst, ew):
    mesh = plsc.VectorSubcoreMesh(core_axis_name="c", subcore_axis_name="s")
    f = pl.kernel(
        _sc_body,
        out_type=(
            jax.ShapeDtypeStruct((NC * N * N,), jnp.float32),
            jax.ShapeDtypeStruct((NC * 2 * N,), jnp.float32),
        ),
        mesh=mesh,
        scratch_types=(
            pltpu.VMEM_SHARED((N * N,), jnp.float32),
            pltpu.VMEM_SHARED((N,), jnp.float32),
            pltpu.VMEM_SHARED((N,), jnp.float32),
            pltpu.VMEM((EPW,), jnp.int32),
            pltpu.VMEM((EPW,), jnp.int32),
            pltpu.VMEM((EPW,), jnp.float32),
            pltpu.VMEM((ROWS, 128), jnp.int32),
            pltpu.VMEM((ROWS, 128), jnp.int32),
            pltpu.VMEM((ROWS, 128), jnp.int32),
            pltpu.VMEM((128,), jnp.float32),
            pltpu.VMEM((8192,), jnp.float32),
            pltpu.SemaphoreType.DMA,
        ),
    )
    return f(src, dst, ew)


def _tc_body(ap_ref, din_ref, dout_ref, feat_ref, w1_ref, b1_ref, wgt_ref,
             l1w_ref, l1b_ref, l2w_ref, l2b_ref, l3w_ref, l3b_ref, out_ref):
    ri = lax.rsqrt(jnp.maximum(din_ref[0] + din_ref[1], 1.0))
    ro = lax.rsqrt(jnp.maximum(dout_ref[0] + dout_ref[1], 1.0))

    dot = functools.partial(jnp.dot, preferred_element_type=jnp.float32)

    m_blocks = []
    for u in range(8):
        a_u = (ap_ref[0, :, u, :, :] + ap_ref[1, :, u, :, :]).reshape(N, 128)
        m_blocks.append(a_u * ri * ro[:, u * 128:(u + 1) * 128])

    t0 = sum(dot(m_blocks[u], feat_ref[u * 128:(u + 1) * 128, :])
             for u in range(8))
    x1 = jnp.maximum(dot(t0, w1_ref[...]) + b1_ref[...], 0.0)
    t1 = dot(x1, wgt_ref[...])
    x2 = jnp.maximum(
        sum(dot(m_blocks[u], t1[u * 128:(u + 1) * 128, :]) for u in range(8)),
        0.0)
    x3 = jnp.maximum(dot(x2, l1w_ref[...]) + l1b_ref[...], 0.0)
    x4 = jnp.maximum(dot(x3, l2w_ref[...]) + l2b_ref[...], 0.0)
    out_ref[...] = dot(x4, l3w_ref[...]) + l3b_ref[...]


def kernel(g, features, weight, edge_weight, W1, b1, lin1_W, lin1_b,
           lin2_W, lin2_b, lin3_W, lin3_b):
    a_flat, deg_flat = _sc_build(g[0], g[1], edge_weight)
    ap = a_flat.reshape(NC, N // 8, 8, 8, 128)
    degs = deg_flat.reshape(NC, 2, N)
    din = degs[:, 0, :].reshape(NC, N, 1)
    dout = degs[:, 1, :].reshape(NC, 1, N)

    out = pl.pallas_call(
        _tc_body,
        out_shape=jax.ShapeDtypeStruct((N, 16), jnp.float32),
    )(ap, din, dout, features, W1, b1.reshape(1, -1), weight,
      lin1_W, lin1_b.reshape(1, -1), lin2_W, lin2_b.reshape(1, -1),
      lin3_W, lin3_b.reshape(1, -1))
    return out
